# baseline (speedup 1.0000x reference)
import functools

import jax
import jax.numpy as jnp
from jax import lax
from jax.experimental import pallas as pl
from jax.experimental.pallas import tpu as pltpu
from jax.experimental.pallas import tpu_sc as plsc

N = 10000
DEG = 32
IN_DIM = 128
OUT_DIM = 128

NC = 2
NS = 16
CHUNK = 4
NBUF = 2
SPAN = 640
A_NODES = 536
B_NODES = SPAN - A_NODES
A_CHUNKS = A_NODES // CHUNK
B_CHUNKS = B_NODES // CHUNK
NPAD = NS * SPAN
VPR = IN_DIM // 16


def _sc_gather_sum(x, idx_all):
    mesh = plsc.VectorSubcoreMesh(core_axis_name="c", subcore_axis_name="s")

    @functools.partial(
        pl.kernel,
        out_type=jax.ShapeDtypeStruct((NPAD, IN_DIM), jnp.float32),
        mesh=mesh,
        scratch_types=[
            pltpu.VMEM((A_CHUNKS, CHUNK * DEG), jnp.int32),
            pltpu.VMEM((NBUF, CHUNK * DEG, IN_DIM), jnp.float32),
            pltpu.VMEM((A_NODES, IN_DIM), jnp.float32),
            [pltpu.SemaphoreType.DMA] * NBUF,
        ],
    )
    def gather_sum(x_hbm, idx_hbm, out_hbm, idx_v, stage_v, res_v, sems):
        cid = lax.axis_index("c")
        sid = lax.axis_index("s")
        base = sid * SPAN + cid * A_NODES
        pltpu.sync_copy(idx_hbm.at[cid, sid], idx_v)

        def fire(c, b):
            pltpu.async_copy(x_hbm.at[idx_v.at[c]], stage_v.at[b], sems[b])

        def drain(c, b):
            pltpu.make_async_copy(
                x_hbm.at[idx_v.at[c]], stage_v.at[b], sems[b]
            ).wait()

        def run(nchunks):
            for b in range(NBUF):
                fire(b, b)

            @pl.loop(0, nchunks, step=NBUF)
            def _(c):
                for b in range(NBUF):
                    cur = c + b
                    drain(cur, b)
                    stage = stage_v.at[b]
                    for n in range(CHUNK):
                        zero = jnp.zeros((16,), jnp.float32)

                        @pl.loop(0, DEG, init_carry=(zero,) * VPR, unroll=4)
                        def acc(r, carry, stage=stage, n=n):
                            return tuple(
                                carry[v] + stage[n * DEG + r, pl.ds(v * 16, 16)]
                                for v in range(VPR)
                            )

                        for v in range(VPR):
                            res_v[cur * CHUNK + n, pl.ds(v * 16, 16)] = acc[v]

                    nxt = cur + NBUF

                    @pl.when(nxt < nchunks)
                    def _(nxt=nxt, b=b):
                        fire(nxt, b)

        @pl.when(cid == 0)
        def _():
            run(A_CHUNKS)
            pltpu.sync_copy(res_v, out_hbm.at[pl.ds(base, A_NODES)])

        @pl.when(cid == 1)
        def _():
            run(B_CHUNKS)
            pltpu.sync_copy(
                res_v.at[pl.ds(0, B_NODES)], out_hbm.at[pl.ds(base, B_NODES)]
            )

    return gather_sum(x, idx_all)


def _tc_self_body(x_ref, wst_ref, b_ref, o_ref):
    o_ref[...] = (
        jnp.dot(x_ref[...], wst_ref[...], preferred_element_type=jnp.float32)
        + b_ref[...]
    )


def _tc_self(x, wst, bias2d):
    blk = 1000
    grid = N // blk
    return pl.pallas_call(
        _tc_self_body,
        grid=(grid,),
        in_specs=[
            pl.BlockSpec((blk, IN_DIM), lambda i: (i, 0)),
            pl.BlockSpec((IN_DIM, OUT_DIM), lambda i: (0, 0)),
            pl.BlockSpec((1, OUT_DIM), lambda i: (0, 0)),
        ],
        out_specs=pl.BlockSpec((blk, OUT_DIM), lambda i: (i, 0)),
        out_shape=jax.ShapeDtypeStruct((N, OUT_DIM), jnp.float32),
    )(x, wst, bias2d)


def _tc_neigh_body(h_ref, s_ref, wnt_ref, o_ref):
    h = h_ref[...] + jnp.dot(
        s_ref[...], wnt_ref[...], preferred_element_type=jnp.float32
    )
    o_ref[...] = jnp.maximum(h, 0.0)


def _tc_neigh(h_pre, neigh_sum, wnt):
    blk = 1000
    grid = N // blk
    return pl.pallas_call(
        _tc_neigh_body,
        grid=(grid,),
        in_specs=[
            pl.BlockSpec((blk, OUT_DIM), lambda i: (i, 0)),
            pl.BlockSpec((blk, IN_DIM), lambda i: (i, 0)),
            pl.BlockSpec((IN_DIM, OUT_DIM), lambda i: (0, 0)),
        ],
        out_specs=pl.BlockSpec((blk, OUT_DIM), lambda i: (i, 0)),
        out_shape=jax.ShapeDtypeStruct((N, OUT_DIM), jnp.float32),
    )(h_pre, neigh_sum, wnt)


@jax.jit
def kernel(X, neighbors, W_self, W_neigh, bias):
    idx = neighbors.astype(jnp.int32)
    idx = jnp.pad(idx, ((0, NPAD - N), (0, 0)))
    by_span = idx.reshape(NS, SPAN * DEG)
    sc0_idx = by_span[:, : A_NODES * DEG].reshape(NS, A_CHUNKS, CHUNK * DEG)
    sc1_idx = jnp.pad(
        by_span[:, A_NODES * DEG :].reshape(NS, B_CHUNKS, CHUNK * DEG),
        ((0, 0), (0, A_CHUNKS - B_CHUNKS), (0, 0)),
    )
    idx_all = jnp.stack([sc0_idx, sc1_idx])
    neigh_sum = _sc_gather_sum(X, idx_all)
    h_pre = _tc_self(X, W_self.T, bias.reshape(1, OUT_DIM))
    return _tc_neigh(h_pre, neigh_sum, W_neigh.T)
